# Initial kernel scaffold; baseline (speedup 1.0000x reference)
import jax
import jax.numpy as jnp
from jax.experimental import pallas as pl


def kernel(predicted_attn, token_timestamps, attention_mask):
    raise NotImplementedError("write your pallas kernel here")


# speedup vs baseline: 1.5540x; 1.5540x over previous
import jax
import jax.numpy as jnp
from jax import lax
from jax.experimental import pallas as pl
from jax.experimental.pallas import tpu as pltpu

FRAME_RATE = 12.5
INTERP_FRAMES = 4


def _loss_body(pred_ref, ts_ref, mask_ref, out_ref, acc_ref):
    i = pl.program_id(0)
    nb = pl.num_programs(0)

    @pl.when(i == 0)
    def _init():
        acc_ref[0] = 0.0
        acc_ref[1] = 0.0

    pred = pred_ref[...]
    Tt, F = pred.shape
    ts = ts_ref[0]
    start = ts[:, 0:1]
    end = ts[:, 1:2]

    sf = jnp.clip(jnp.floor(start * FRAME_RATE), 0.0, float(F - 1))
    ef0 = jnp.floor(end * FRAME_RATE)
    ef = jnp.maximum(sf + 1.0, jnp.minimum(ef0 + 1.0, float(F)))

    frames = lax.broadcasted_iota(jnp.int32, (Tt, F), 1).astype(jnp.float32)
    w = jnp.minimum(frames - (sf - 5.0), (ef + 4.0) - frames)
    w = jnp.clip(w, 0.0, 5.0)

    dot = jnp.sum(pred * w, axis=-1) * 0.2
    psq = jnp.sum(pred * pred, axis=-1)
    gsq = jnp.sum(w * w, axis=-1) * 0.04

    pn = jnp.maximum(jnp.sqrt(psq), 1e-8)
    gn = jnp.maximum(jnp.sqrt(gsq), 1e-8)
    cos = dot / (pn * gn)

    m = mask_ref[0, 0]
    num = jnp.sum((1.0 - cos) * m)
    den = jnp.sum(m)

    acc_ref[0] += num
    acc_ref[1] += den

    @pl.when(i == nb - 1)
    def _fin():
        out_ref[0, 0] = acc_ref[0] / jnp.maximum(acc_ref[1], 1.0)


def kernel(predicted_attn, token_timestamps, attention_mask):
    B, T, F = predicted_attn.shape
    N = B * T
    Tt = 128
    NB = N // Tt

    pred = predicted_attn.reshape(N, F)
    ts = token_timestamps.reshape(NB, Tt, 2)
    mask = attention_mask.astype(jnp.float32).reshape(NB, 1, Tt)

    out = pl.pallas_call(
        _loss_body,
        grid=(NB,),
        in_specs=[
            pl.BlockSpec((Tt, F), lambda i: (i, 0)),
            pl.BlockSpec((1, Tt, 2), lambda i: (i, 0, 0)),
            pl.BlockSpec((1, 1, Tt), lambda i: (i, 0, 0)),
        ],
        out_specs=pl.BlockSpec(memory_space=pltpu.SMEM),
        out_shape=jax.ShapeDtypeStruct((1, 1), jnp.float32),
        scratch_shapes=[pltpu.SMEM((2,), jnp.float32)],
        compiler_params=pltpu.CompilerParams(
            dimension_semantics=("arbitrary",),
        ),
    )(pred, ts, mask)
    return out[0, 0]
